# baseline (speedup 1.0000x reference)
import dataclasses
import functools

import jax
import jax.numpy as jnp
from jax import lax
from jax.experimental import pallas as pl
from jax.experimental.pallas import tpu as pltpu
from jax.experimental.pallas import tpu_sc as plsc

_VOCAB = 1000000
_EMB = 32
_SEQ = 200
_BATCH = 4096
_NW = 32
_BPW = _BATCH // _NW
_CH = 10
_RING = 2 * _CH
_NCH = _SEQ // _CH

_mesh = plsc.VectorSubcoreMesh(core_axis_name="c", subcore_axis_name="s")

_cp = pltpu.CompilerParams()
for _f, _v in (("needs_layout_passes", False), ("use_tc_tiling_on_sc", False)):
    if _f in pltpu.CompilerParams.__dataclass_fields__:
        _cp = dataclasses.replace(_cp, **{_f: _v})


@functools.partial(
    pl.kernel,
    out_type=jax.ShapeDtypeStruct((_BATCH,), jnp.float32),
    mesh=_mesh,
    compiler_params=_cp,
    scratch_types=[
        pltpu.VMEM((_SEQ, _BPW), jnp.int32),
        pltpu.VMEM((_RING, _BPW, _EMB), jnp.float32),
        pltpu.VMEM((_BPW,), jnp.float32),
        pltpu.VMEM((_BPW * 16,), jnp.float32),
        pltpu.VMEM((40,), jnp.float32),
        pltpu.SemaphoreType.DMA,
        pltpu.SemaphoreType.DMA,
    ],
)
def _sc_fused(text_hbm, table_hbm, fcp_hbm, out_hbm,
              idx_v, ring_v, res_v, acc_f, fc_s, sem_a, sem_b):
    wid = lax.axis_index("s") * 2 + lax.axis_index("c")
    base = wid * _BPW
    pltpu.sync_copy(fcp_hbm, fc_s)
    pltpu.sync_copy(text_hbm.at[:, pl.ds(base, _BPW)], idx_v)

    iota = lax.iota(jnp.int32, 16)

    def fire(s, slot, sem):
        pltpu.async_copy(table_hbm.at[idx_v.at[s]], ring_v.at[slot], sem)

    def fire_chunk(c, half, sem):
        for j in range(_CH):
            fire(c * _CH + j, half + j, sem)

    def drain_chunk(half, sem):
        for j in range(_CH):
            pltpu.make_async_copy(
                table_hbm.at[idx_v.at[j]], ring_v.at[half + j], sem,
            ).wait()

    fcA = fc_s[pl.ds(0, 16)]
    fcB = fc_s[pl.ds(16, 16)]
    zero16 = jnp.zeros((16,), jnp.float32)

    def zero_body(i, _):
        acc_f[pl.ds(i * 16, 16)] = zero16
        return 0

    lax.fori_loop(0, _BPW, zero_body, 0)

    def acc_chunk(half):
        def b_body(b4, _):
            for u in range(8):
                b = b4 * 8 + u
                a = acc_f[pl.ds(b * 16, 16)]
                for j in range(_CH):
                    vA = ring_v[half + j, b, pl.ds(0, 16)]
                    vB = ring_v[half + j, b, pl.ds(16, 16)]
                    a = a + vA * fcA + vB * fcB
                acc_f[pl.ds(b * 16, 16)] = a
            return 0

        lax.fori_loop(0, _BPW // 8, b_body, 0)

    fire_chunk(0, 0, sem_a)

    def pair_body(k, _):
        fire_chunk(2 * k + 1, _CH, sem_b)
        drain_chunk(0, sem_a)
        acc_chunk(0)

        @pl.when(k + 1 < _NCH // 2)
        def _():
            fire_chunk(2 * k + 2, 0, sem_a)

        drain_chunk(_CH, sem_b)
        acc_chunk(_CH)
        return 0

    lax.fori_loop(0, _NCH // 2, pair_body, 0)

    bias = plsc.load_gather(fc_s, [jnp.full((16,), _EMB, jnp.int32)])
    for g in range(8):
        a = zero16
        for l in range(16):
            a = a + plsc.load_gather(acc_f, [(iota + g * 16) * 16 + l])
        res_v[pl.ds(g * 16, 16)] = a * (1.0 / _SEQ) + bias
    pltpu.sync_copy(res_v, out_hbm.at[pl.ds(base, _BPW)])


def kernel(text, table, fc_w, fc_b):
    fcp = jnp.zeros((40,), jnp.float32)
    fcp = fcp.at[:_EMB].set(fc_w.reshape(-1).astype(jnp.float32))
    fcp = fcp.at[_EMB].set(fc_b.reshape(())[()].astype(jnp.float32))
    out = _sc_fused(text, table, fcp)
    return out.reshape(_BATCH, 1)


# speedup vs baseline: 1.1316x; 1.1316x over previous
import functools

import jax
import jax.numpy as jnp
from jax import lax
from jax.experimental import pallas as pl
from jax.experimental.pallas import tpu as pltpu
from jax.experimental.pallas import tpu_sc as plsc

_VOCAB = 1000000
_EMB = 32
_SEQ = 200
_BATCH = 4096
_NW = 32
_BPW = _BATCH // _NW
_RB = 8192


def _tv_body(t_ref, w_ref, b_ref, o_ref):
    tb = t_ref[...].astype(jnp.bfloat16)
    d = lax.dot_general(w_ref[...], tb, (((1,), (1,)), ((), ())),
                        preferred_element_type=jnp.float32)
    o_ref[...] = d[0] + b_ref[0]


def _compute_tv(table, w, fc_b):
    return pl.pallas_call(
        _tv_body,
        grid=(pl.cdiv(_VOCAB, _RB),),
        in_specs=[
            pl.BlockSpec((_RB, _EMB), lambda i: (i, 0)),
            pl.BlockSpec((4, _EMB), lambda i: (0, 0)),
            pl.BlockSpec(memory_space=pltpu.SMEM),
        ],
        out_specs=pl.BlockSpec((_RB,), lambda i: (i,)),
        out_shape=jax.ShapeDtypeStruct((_VOCAB,), jnp.float32),
    )(table, w, fc_b)


_CH = 20

_mesh = plsc.VectorSubcoreMesh(core_axis_name="c", subcore_axis_name="s")


@functools.partial(
    pl.kernel,
    out_type=jax.ShapeDtypeStruct((_BATCH,), jnp.float32),
    mesh=_mesh,
    scratch_types=[
        pltpu.VMEM((_SEQ, _BPW), jnp.int32),
        pltpu.VMEM((_SEQ, _BPW), jnp.float32),
        pltpu.VMEM((_BPW,), jnp.float32),
        pltpu.SemaphoreType.DMA,
    ],
)
def _sc_pool(text_hbm, tv_hbm, out_hbm, idx_v, val_v, res_v, sem):
    wid = lax.axis_index("s") * 2 + lax.axis_index("c")
    base = wid * _BPW
    pltpu.sync_copy(text_hbm.at[:, pl.ds(base, _BPW)], idx_v)

    @pl.loop(0, _SEQ, step=_CH)
    def _gather(s0):
        cps = [
            pltpu.async_copy(tv_hbm.at[idx_v.at[s0 + j]], val_v.at[s0 + j], sem)
            for j in range(_CH)
        ]
        for cp in cps:
            cp.wait()

    def _acc_body(s, accs):
        return tuple(accs[j] + val_v[s, pl.ds(j * 16, 16)] for j in range(8))

    accs = lax.fori_loop(
        0, _SEQ, _acc_body,
        tuple(jnp.zeros((16,), jnp.float32) for _ in range(8)),
    )
    for j in range(8):
        res_v[pl.ds(j * 16, 16)] = accs[j] * (1.0 / _SEQ)
    pltpu.sync_copy(res_v, out_hbm.at[pl.ds(base, _BPW)])


def kernel(text, table, fc_w, fc_b):
    w = jnp.broadcast_to(fc_w.reshape(1, _EMB), (4, _EMB)).astype(jnp.bfloat16)
    tv = _compute_tv(table, w, fc_b)
    out = _sc_pool(text, tv)
    return out.reshape(_BATCH, 1)
